# baseline (speedup 1.0000x reference)
import jax
import jax.numpy as jnp
import numpy as np
from jax import lax
from jax.experimental import pallas as pl
from jax.experimental.pallas import tpu as pltpu
from jax.experimental.pallas import tpu_sc as plsc

N_NODES = 10000
N_PAD = 10240
N_SUBCORES = 16
EDGE_BATCH = 128
ROWS_PER_TILE = N_PAD // N_SUBCORES
IC = 40


DEG_RED_CHUNK = 160


def _make_sc_agg(n_idx_rows, fh, n_pass, nbuf, with_deg, ic):
    rpt = n_idx_rows // N_SUBCORES
    nzc = ROWS_PER_TILE // EDGE_BATCH
    if rpt % ic:
        ic = 8
    n_ic = rpt // ic
    lead = nbuf // 2
    n_tab = 2 * n_pass

    out_type = [jax.ShapeDtypeStruct((n_tab, N_PAD, fh), jnp.float32)]
    scratch = (
        [pltpu.VMEM((ic, EDGE_BATCH), jnp.int32)] * 2
        + [pltpu.VMEM((EDGE_BATCH, fh // 2), jnp.int32)] * nbuf
        + [pltpu.VMEM((EDGE_BATCH, fh), jnp.float32)] * nbuf
        + [pltpu.VMEM_SHARED((N_PAD, fh), jnp.float32)]
        + [pltpu.SemaphoreType.DMA] * (2 * nbuf)
    )
    if with_deg:
        out_type.append(jax.ShapeDtypeStruct((N_PAD,), jnp.float32))
        scratch += [
            pltpu.VMEM((N_PAD,), jnp.float32),
            pltpu.VMEM_SHARED((N_SUBCORES, N_PAD), jnp.float32),
            pltpu.VMEM((N_SUBCORES, DEG_RED_CHUNK), jnp.float32),
            pltpu.VMEM((ROWS_PER_TILE,), jnp.float32),
        ]

    mesh = plsc.VectorSubcoreMesh(core_axis_name="c", subcore_axis_name="s")

    def body(*refs):
        tabs = refs[:n_tab]
        srcr, dstr, z2dr = refs[n_tab:n_tab + 3]
        k = n_tab + 3 + (1 if with_deg else 0)
        outs = refs[k:k + len(out_type)]
        aggr = outs[0]
        sc = refs[k + len(out_type):]
        srcv, dstv = sc[0], sc[1]
        gbufs = sc[2:2 + nbuf]
        ubufs = sc[2 + nbuf:2 + 2 * nbuf]
        aggsh = sc[2 + 2 * nbuf]
        gsems = sc[3 + 2 * nbuf:3 + 3 * nbuf]
        ssems = sc[3 + 3 * nbuf:3 + 4 * nbuf]
        if with_deg:
            z1dr = refs[n_tab + 3]
            degr = outs[1]
            hist, grid, red, degv = sc[3 + 4 * nbuf:]

        c = lax.axis_index("c")
        s = lax.axis_index("s")
        row0 = s * ROWS_PER_TILE
        r0 = s * rpt
        ones = jnp.full((16,), 1.0, jnp.float32)
        if with_deg:
            pltpu.sync_copy(z1dr, hist)

        def run_pass(h, slab, deg_pass):
            for i in range(nzc):
                pltpu.sync_copy(
                    z2dr, aggsh.at[pl.ds(row0 + i * EDGE_BATCH, EDGE_BATCH)])
            plsc.subcore_barrier()

            def outer(ci, carry):
                pltpu.sync_copy(srcr.at[pl.ds(r0 + ci * ic, ic)], srcv)
                pltpu.sync_copy(dstr.at[pl.ds(r0 + ci * ic, ic)], dstv)
                gd, sd = {}, {}
                for r in range(lead):
                    gd[r % nbuf] = pltpu.async_copy(
                        h.at[srcv.at[r]], gbufs[r % nbuf], gsems[r % nbuf])
                for j in range(ic):
                    b = j % nbuf
                    gd.pop(b).wait()

                    def conv(r, carry2, b=b):
                        for blk in range(fh // 32):
                            w = gbufs[b][r, pl.ds(blk * 16, 16)]
                            lo = plsc.bitcast(w << 16, jnp.float32)
                            hi = plsc.bitcast(w & jnp.int32(-65536),
                                              jnp.float32)
                            ubufs[b][r, pl.ds(blk * 32, 16)] = lo
                            ubufs[b][r, pl.ds(blk * 32 + 16, 16)] = hi
                        return carry2

                    lax.fori_loop(0, EDGE_BATCH, conv, 0)
                    sd[b] = pltpu.async_copy(
                        ubufs[b], aggsh.at[dstv.at[j]], ssems[b], add=True)
                    if deg_pass:
                        for q in range(EDGE_BATCH // 16):
                            idx = dstv[j, pl.ds(q * 16, 16)]
                            plsc.addupdate_scatter(hist, (idx,), ones)
                    jn = j + lead
                    if jn < ic:
                        bn = jn % nbuf
                        if bn in sd:
                            sd.pop(bn).wait()
                        gd[bn] = pltpu.async_copy(
                            h.at[srcv.at[jn]], gbufs[bn], gsems[bn])
                for b in sorted(sd):
                    sd[b].wait()
                return carry

            lax.fori_loop(0, n_ic, outer, 0)
            plsc.subcore_barrier()
            pltpu.sync_copy(
                aggsh.at[pl.ds(row0, ROWS_PER_TILE)],
                aggr.at[slab, pl.ds(row0, ROWS_PER_TILE)],
            )

        for p in range(n_pass):
            dp = with_deg and p == 0

            @pl.when(c == 0)
            def _(p=p, dp=dp):
                run_pass(tabs[p], p, dp)

            @pl.when(c != 0)
            def _(p=p, dp=dp):
                run_pass(tabs[n_pass + p], n_pass + p, dp)

        if with_deg:
            pltpu.sync_copy(hist, grid.at[s])
            plsc.subcore_barrier()
            for t in range(ROWS_PER_TILE // DEG_RED_CHUNK):
                pltpu.sync_copy(
                    grid.at[:, pl.ds(row0 + t * DEG_RED_CHUNK, DEG_RED_CHUNK)],
                    red)

                def rstep(kk, carry, t=t):
                    acc = jnp.zeros((16,), jnp.float32)
                    for r in range(N_SUBCORES):
                        acc = acc + red[r, pl.ds(kk * 16, 16)]
                    degv[pl.ds(t * DEG_RED_CHUNK + kk * 16, 16)] = acc
                    return carry

                lax.fori_loop(0, DEG_RED_CHUNK // 16, rstep, 0)

            @pl.when(c == 0)
            def _():
                pltpu.sync_copy(degv, degr.at[pl.ds(row0, ROWS_PER_TILE)])

    return pl.kernel(
        body, out_type=out_type, mesh=mesh, scratch_types=scratch,
        compiler_params=pltpu.CompilerParams(
            needs_layout_passes=False, use_tc_tiling_on_sc=False))


def _dot(a, b):
    return jnp.dot(a, b, preferred_element_type=jnp.float32)


def _tc_layer(h_parts, agg, deg2d, ws, wn, b2d, n_out_parts, wnx=None,
              relu=True, add_agg=False, out_dtype=jnp.float32,
              wnx_dtype=jnp.float32):
    ns, np_, fa = agg.shape
    dps = [h.shape[1] for h in h_parts]
    dout = ws.shape[1]
    bm = 256
    nblk = np_ // bm
    nh = len(h_parts)

    def bodyfn(*refs):
        hs = refs[:nh]
        ag, dg, wsr = refs[nh], refs[nh + 1], refs[nh + 2]
        k = nh + 3
        if not add_agg:
            wnr = refs[k]
            k += 1
        br = refs[k]
        k += 1
        if wnx is not None:
            wxr = refs[k]
            k += 1
        outs = refs[k:]
        inv = 1.0 / jnp.maximum(dg[...], 1.0)
        off = 0
        acc = br[...] * jnp.ones((bm, 1), jnp.float32)
        for i, dp in enumerate(dps):
            acc += _dot(hs[i][...].astype(jnp.float32), wsr[pl.ds(off, dp), :])
            off += dp
        if add_agg:
            acc += jnp.concatenate([ag[i] for i in range(ns)], axis=1) * inv
        else:
            for i in range(ns):
                acc += _dot(ag[i] * inv, wnr[pl.ds(i * fa, fa), :])
        if relu:
            acc = jnp.maximum(acc, 0.0)
        po = dout // n_out_parts
        for i in range(n_out_parts):
            outs[i][...] = acc[:, i * po:(i + 1) * po].astype(out_dtype)
        if wnx is not None:
            nxt = _dot(acc, wxr[...])
            dx = nxt.shape[1] // 2
            outs[n_out_parts][...] = nxt[:, :dx].astype(wnx_dtype)
            outs[n_out_parts + 1][...] = nxt[:, dx:].astype(wnx_dtype)

    in_specs = [pl.BlockSpec((bm, dp), lambda i: (i, 0)) for dp in dps]
    in_specs += [
        pl.BlockSpec((ns, bm, fa), lambda i: (0, i, 0)),
        pl.BlockSpec((bm, 1), lambda i: (i, 0)),
        pl.BlockSpec((sum(dps), dout), lambda i: (0, 0)),
    ]
    args = list(h_parts) + [agg, deg2d, ws]
    if not add_agg:
        in_specs.append(pl.BlockSpec((ns * fa, dout), lambda i: (0, 0)))
        args.append(wn)
    in_specs.append(pl.BlockSpec((1, dout), lambda i: (0, 0)))
    args.append(b2d)
    po = dout // n_out_parts
    out_shape = [jax.ShapeDtypeStruct((np_, po), out_dtype)] * n_out_parts
    out_specs = [pl.BlockSpec((bm, po), lambda i: (i, 0))] * n_out_parts
    if wnx is not None:
        dx = wnx.shape[1]
        in_specs.append(pl.BlockSpec((dout, dx), lambda i: (0, 0)))
        args.append(wnx)
        out_shape += [jax.ShapeDtypeStruct((np_, dx // 2), wnx_dtype)] * 2
        out_specs += [pl.BlockSpec((bm, dx // 2), lambda i: (i, 0))] * 2
    return pl.pallas_call(
        bodyfn,
        grid=(nblk,),
        in_specs=in_specs,
        out_specs=out_specs,
        out_shape=out_shape,
    )(*args)


def _uinv(width):
    u = np.empty(width, np.int64)
    for b0 in range(0, width, 32):
        for i in range(16):
            u[b0 + i] = b0 + 2 * i
            u[b0 + 16 + i] = b0 + 2 * i + 1
    return np.argsort(u)


def kernel(x, edge_index, W_self_0, W_neigh_0, b_0, W_self_1, W_neigh_1, b_1,
           W_self_2, W_neigh_2, b_2):
    n, fin = x.shape
    e = edge_index.shape[1]
    quant = N_SUBCORES * EDGE_BATCH * 8
    ep = (e + quant - 1) // quant * quant
    src = jnp.concatenate(
        [edge_index[0], jnp.zeros((ep - e,), jnp.int32)]).reshape(-1, EDGE_BATCH)
    dst = jnp.concatenate(
        [edge_index[1], jnp.full((ep - e,), N_NODES, jnp.int32)]).reshape(-1, EDGE_BATCH)
    n_idx_rows = ep // EDGE_BATCH

    z1d = jnp.zeros((N_PAD,), jnp.float32)
    z64 = jnp.zeros((EDGE_BATCH, 64), jnp.float32)
    xp = jnp.pad(x, ((0, N_PAD - n), (0, 0)))
    x0 = xp[:, :64]
    x1 = xp[:, 64:]
    ui64 = _uinv(64)

    def _pairs(a):
        w = a.shape[1]
        return jax.lax.bitcast_convert_type(
            a.reshape(N_PAD, w // 2, 2), jnp.int32)

    def _tab(a):
        w = a.shape[1]
        return _pairs(a.reshape(N_PAD, w // 32, 2, 16).swapaxes(2, 3)
                      .reshape(N_PAD, w).astype(jnp.bfloat16))

    x0t = _tab(x0)
    x1t = _tab(x1)

    agg0, deg = _make_sc_agg(n_idx_rows, 64, 1, 4, True, 40)(
        x0t, x1t, src, dst, z64, z1d)
    deg2d = deg[:, None]
    cm256 = np.concatenate([64 * qq + ui64 for qq in range(4)])
    q = _tc_layer([x0, x1], agg0, deg2d, W_self_0[:, cm256],
                  W_neigh_0[:, cm256], b_0[cm256][None, :], 4,
                  out_dtype=jnp.bfloat16)

    (agg1,) = _make_sc_agg(n_idx_rows, 64, 2, 6, False, 40)(
        _pairs(q[0]), _pairs(q[1]), _pairs(q[2]), _pairs(q[3]), src, dst, z64)
    d2p = 64
    wn2p = jnp.pad(W_neigh_2, ((0, 0), (0, d2p - W_neigh_2.shape[1])))
    h2a, h2b, hwa, hwb = _tc_layer(
        q, agg1, deg2d, W_self_1[cm256, :], W_neigh_1, b_1[None, :], 2,
        wnx=wn2p[:, ui64], wnx_dtype=jnp.bfloat16)

    (agg2,) = _make_sc_agg(n_idx_rows, 32, 1, 8, False, 40)(
        _pairs(hwa), _pairs(hwb), src, dst,
        jnp.zeros((EDGE_BATCH, 32), jnp.float32))
    ws2p = jnp.pad(W_self_2, ((0, 0), (0, d2p - W_self_2.shape[1])))
    b2p = jnp.pad(b_2, (0, d2p - b_2.shape[0]))[None, :]
    out = _tc_layer([h2a, h2b], agg2, deg2d, ws2p, None, b2p, 1,
                    relu=False, add_agg=True)[0]
    return out[:N_NODES, : W_self_2.shape[1]]


# speedup vs baseline: 1.0192x; 1.0192x over previous
import jax
import jax.numpy as jnp
import numpy as np
from jax import lax
from jax.experimental import pallas as pl
from jax.experimental.pallas import tpu as pltpu
from jax.experimental.pallas import tpu_sc as plsc

N_NODES = 10000
N_PAD = 10240
N_SUBCORES = 16
EDGE_BATCH = 128
ROWS_PER_TILE = N_PAD // N_SUBCORES
IC = 40


DEG_RED_CHUNK = 160


def _make_sc_agg(n_idx_rows, fh, n_pass, nbuf, with_deg, ic):
    rpt = n_idx_rows // N_SUBCORES
    nzc = ROWS_PER_TILE // EDGE_BATCH
    if rpt % ic:
        ic = 8
    n_ic = rpt // ic
    lead = nbuf // 2
    n_tab = 2 * n_pass

    out_type = [jax.ShapeDtypeStruct((n_tab, N_PAD, fh), jnp.float32)]
    scratch = (
        [pltpu.VMEM((ic, EDGE_BATCH), jnp.int32)] * 2
        + [pltpu.VMEM((EDGE_BATCH, fh), jnp.bfloat16)] * nbuf
        + [pltpu.VMEM((EDGE_BATCH, fh), jnp.float32)] * nbuf
        + [pltpu.VMEM_SHARED((N_PAD, fh), jnp.float32)]
        + [pltpu.SemaphoreType.DMA] * (2 * nbuf)
    )
    if with_deg:
        out_type.append(jax.ShapeDtypeStruct((N_PAD,), jnp.float32))
        scratch += [
            pltpu.VMEM((N_PAD,), jnp.float32),
            pltpu.VMEM_SHARED((N_SUBCORES, N_PAD), jnp.float32),
            pltpu.VMEM((N_SUBCORES, DEG_RED_CHUNK), jnp.float32),
            pltpu.VMEM((ROWS_PER_TILE,), jnp.float32),
        ]

    mesh = plsc.VectorSubcoreMesh(core_axis_name="c", subcore_axis_name="s")

    def body(*refs):
        tabs = refs[:n_tab]
        srcr, dstr, z2dr = refs[n_tab:n_tab + 3]
        k = n_tab + 3 + (1 if with_deg else 0)
        outs = refs[k:k + len(out_type)]
        aggr = outs[0]
        sc = refs[k + len(out_type):]
        srcv, dstv = sc[0], sc[1]
        gbufs = sc[2:2 + nbuf]
        ubufs = sc[2 + nbuf:2 + 2 * nbuf]
        aggsh = sc[2 + 2 * nbuf]
        gsems = sc[3 + 2 * nbuf:3 + 3 * nbuf]
        ssems = sc[3 + 3 * nbuf:3 + 4 * nbuf]
        if with_deg:
            z1dr = refs[n_tab + 3]
            degr = outs[1]
            hist, grid, red, degv = sc[3 + 4 * nbuf:]

        c = lax.axis_index("c")
        s = lax.axis_index("s")
        row0 = s * ROWS_PER_TILE
        r0 = s * rpt
        ones = jnp.full((16,), 1.0, jnp.float32)
        if with_deg:
            pltpu.sync_copy(z1dr, hist)

        def run_pass(h, slab, deg_pass):
            for i in range(nzc):
                pltpu.sync_copy(
                    z2dr, aggsh.at[pl.ds(row0 + i * EDGE_BATCH, EDGE_BATCH)])
            plsc.subcore_barrier()

            def outer(ci, carry):
                pltpu.sync_copy(srcr.at[pl.ds(r0 + ci * ic, ic)], srcv)
                pltpu.sync_copy(dstr.at[pl.ds(r0 + ci * ic, ic)], dstv)
                gd, sd = {}, {}
                for r in range(lead):
                    gd[r % nbuf] = pltpu.async_copy(
                        h.at[srcv.at[r]], gbufs[r % nbuf], gsems[r % nbuf])
                for j in range(ic):
                    b = j % nbuf
                    gd.pop(b).wait()

                    def conv(r, carry2, b=b):
                        for blk in range(fh // 32):
                            w = gbufs[b][r, pl.ds(blk * 32, 32)]
                            lo, hi = plsc.unpack(
                                w, format=plsc.PackFormat.INTERLEAVED)
                            ubufs[b][r, pl.ds(blk * 32, 16)] = lo
                            ubufs[b][r, pl.ds(blk * 32 + 16, 16)] = hi
                        return carry2

                    lax.fori_loop(0, EDGE_BATCH, conv, 0)
                    sd[b] = pltpu.async_copy(
                        ubufs[b], aggsh.at[dstv.at[j]], ssems[b], add=True)
                    if deg_pass:
                        for q in range(EDGE_BATCH // 16):
                            idx = dstv[j, pl.ds(q * 16, 16)]
                            plsc.addupdate_scatter(hist, (idx,), ones)
                    jn = j + lead
                    if jn < ic:
                        bn = jn % nbuf
                        if bn in sd:
                            sd.pop(bn).wait()
                        gd[bn] = pltpu.async_copy(
                            h.at[srcv.at[jn]], gbufs[bn], gsems[bn])
                for b in sorted(sd):
                    sd[b].wait()
                return carry

            lax.fori_loop(0, n_ic, outer, 0)
            plsc.subcore_barrier()
            pltpu.sync_copy(
                aggsh.at[pl.ds(row0, ROWS_PER_TILE)],
                aggr.at[slab, pl.ds(row0, ROWS_PER_TILE)],
            )

        for p in range(n_pass):
            dp = with_deg and p == 0

            @pl.when(c == 0)
            def _(p=p, dp=dp):
                run_pass(tabs[p], p, dp)

            @pl.when(c != 0)
            def _(p=p, dp=dp):
                run_pass(tabs[n_pass + p], n_pass + p, dp)

        if with_deg:
            pltpu.sync_copy(hist, grid.at[s])
            plsc.subcore_barrier()
            for t in range(ROWS_PER_TILE // DEG_RED_CHUNK):
                pltpu.sync_copy(
                    grid.at[:, pl.ds(row0 + t * DEG_RED_CHUNK, DEG_RED_CHUNK)],
                    red)

                def rstep(kk, carry, t=t):
                    acc = jnp.zeros((16,), jnp.float32)
                    for r in range(N_SUBCORES):
                        acc = acc + red[r, pl.ds(kk * 16, 16)]
                    degv[pl.ds(t * DEG_RED_CHUNK + kk * 16, 16)] = acc
                    return carry

                lax.fori_loop(0, DEG_RED_CHUNK // 16, rstep, 0)

            @pl.when(c == 0)
            def _():
                pltpu.sync_copy(degv, degr.at[pl.ds(row0, ROWS_PER_TILE)])

    return pl.kernel(
        body, out_type=out_type, mesh=mesh, scratch_types=scratch,
        compiler_params=pltpu.CompilerParams(
            needs_layout_passes=False, use_tc_tiling_on_sc=False))


def _dot(a, b):
    return jnp.dot(a, b, preferred_element_type=jnp.float32)


def _tc_layer(h_parts, agg, deg2d, ws, wn, b2d, n_out_parts, wnx=None,
              relu=True, add_agg=False, out_dtype=jnp.float32,
              wnx_dtype=jnp.float32):
    ns, np_, fa = agg.shape
    dps = [h.shape[1] for h in h_parts]
    dout = ws.shape[1]
    bm = 256
    nblk = np_ // bm
    nh = len(h_parts)

    def bodyfn(*refs):
        hs = refs[:nh]
        ag, dg, wsr = refs[nh], refs[nh + 1], refs[nh + 2]
        k = nh + 3
        if not add_agg:
            wnr = refs[k]
            k += 1
        br = refs[k]
        k += 1
        if wnx is not None:
            wxr = refs[k]
            k += 1
        outs = refs[k:]
        inv = 1.0 / jnp.maximum(dg[...], 1.0)
        off = 0
        acc = br[...] * jnp.ones((bm, 1), jnp.float32)
        for i, dp in enumerate(dps):
            acc += _dot(hs[i][...].astype(jnp.float32), wsr[pl.ds(off, dp), :])
            off += dp
        if add_agg:
            acc += jnp.concatenate([ag[i] for i in range(ns)], axis=1) * inv
        else:
            for i in range(ns):
                acc += _dot(ag[i] * inv, wnr[pl.ds(i * fa, fa), :])
        if relu:
            acc = jnp.maximum(acc, 0.0)
        po = dout // n_out_parts
        for i in range(n_out_parts):
            outs[i][...] = acc[:, i * po:(i + 1) * po].astype(out_dtype)
        if wnx is not None:
            nxt = _dot(acc, wxr[...])
            dx = nxt.shape[1] // 2
            outs[n_out_parts][...] = nxt[:, :dx].astype(wnx_dtype)
            outs[n_out_parts + 1][...] = nxt[:, dx:].astype(wnx_dtype)

    in_specs = [pl.BlockSpec((bm, dp), lambda i: (i, 0)) for dp in dps]
    in_specs += [
        pl.BlockSpec((ns, bm, fa), lambda i: (0, i, 0)),
        pl.BlockSpec((bm, 1), lambda i: (i, 0)),
        pl.BlockSpec((sum(dps), dout), lambda i: (0, 0)),
    ]
    args = list(h_parts) + [agg, deg2d, ws]
    if not add_agg:
        in_specs.append(pl.BlockSpec((ns * fa, dout), lambda i: (0, 0)))
        args.append(wn)
    in_specs.append(pl.BlockSpec((1, dout), lambda i: (0, 0)))
    args.append(b2d)
    po = dout // n_out_parts
    out_shape = [jax.ShapeDtypeStruct((np_, po), out_dtype)] * n_out_parts
    out_specs = [pl.BlockSpec((bm, po), lambda i: (i, 0))] * n_out_parts
    if wnx is not None:
        dx = wnx.shape[1]
        in_specs.append(pl.BlockSpec((dout, dx), lambda i: (0, 0)))
        args.append(wnx)
        out_shape += [jax.ShapeDtypeStruct((np_, dx // 2), wnx_dtype)] * 2
        out_specs += [pl.BlockSpec((bm, dx // 2), lambda i: (i, 0))] * 2
    return pl.pallas_call(
        bodyfn,
        grid=(nblk,),
        in_specs=in_specs,
        out_specs=out_specs,
        out_shape=out_shape,
    )(*args)


def _uinv(width):
    u = np.empty(width, np.int64)
    for b0 in range(0, width, 32):
        for i in range(16):
            u[b0 + i] = b0 + 2 * i
            u[b0 + 16 + i] = b0 + 2 * i + 1
    return np.argsort(u)


def kernel(x, edge_index, W_self_0, W_neigh_0, b_0, W_self_1, W_neigh_1, b_1,
           W_self_2, W_neigh_2, b_2):
    n, fin = x.shape
    e = edge_index.shape[1]
    quant = N_SUBCORES * EDGE_BATCH * 8
    ep = (e + quant - 1) // quant * quant
    src = jnp.concatenate(
        [edge_index[0], jnp.zeros((ep - e,), jnp.int32)]).reshape(-1, EDGE_BATCH)
    dst = jnp.concatenate(
        [edge_index[1], jnp.full((ep - e,), N_NODES, jnp.int32)]).reshape(-1, EDGE_BATCH)
    n_idx_rows = ep // EDGE_BATCH

    z1d = jnp.zeros((N_PAD,), jnp.float32)
    z64 = jnp.zeros((EDGE_BATCH, 64), jnp.float32)
    xp = jnp.pad(x, ((0, N_PAD - n), (0, 0)))
    x0 = xp[:, :64]
    x1 = xp[:, 64:]
    ui64 = _uinv(64)

    def _tab(a):
        w = a.shape[1]
        return (a.reshape(N_PAD, w // 32, 2, 16).swapaxes(2, 3)
                .reshape(N_PAD, w).astype(jnp.bfloat16))

    x0t = _tab(x0)
    x1t = _tab(x1)

    agg0, deg = _make_sc_agg(n_idx_rows, 64, 1, 4, True, 40)(
        x0t, x1t, src, dst, z64, z1d)
    deg2d = deg[:, None]
    cm256 = np.concatenate([64 * qq + ui64 for qq in range(4)])
    q = _tc_layer([x0, x1], agg0, deg2d, W_self_0[:, cm256],
                  W_neigh_0[:, cm256], b_0[cm256][None, :], 4,
                  out_dtype=jnp.bfloat16)

    (agg1,) = _make_sc_agg(n_idx_rows, 64, 2, 6, False, 40)(
        q[0], q[1], q[2], q[3], src, dst, z64)
    d2p = 64
    wn2p = jnp.pad(W_neigh_2, ((0, 0), (0, d2p - W_neigh_2.shape[1])))
    h2a, h2b, hwa, hwb = _tc_layer(
        q, agg1, deg2d, W_self_1[cm256, :], W_neigh_1, b_1[None, :], 2,
        wnx=wn2p[:, ui64], wnx_dtype=jnp.bfloat16)

    (agg2,) = _make_sc_agg(n_idx_rows, 32, 1, 8, False, 40)(
        hwa, hwb, src, dst, jnp.zeros((EDGE_BATCH, 32), jnp.float32))
    ws2p = jnp.pad(W_self_2, ((0, 0), (0, d2p - W_self_2.shape[1])))
    b2p = jnp.pad(b_2, (0, d2p - b_2.shape[0]))[None, :]
    out = _tc_layer([h2a, h2b], agg2, deg2d, ws2p, None, b2p, 1,
                    relu=False, add_agg=True)[0]
    return out[:N_NODES, : W_self_2.shape[1]]
